# Initial kernel scaffold; baseline (speedup 1.0000x reference)
import jax
import jax.numpy as jnp
from jax.experimental import pallas as pl


def kernel(boxes, scores):
    raise NotImplementedError("write your pallas kernel here")


# speedup vs baseline: 1633.8379x; 1633.8379x over previous
import jax
import jax.numpy as jnp
from jax.experimental import pallas as pl
from jax.experimental.pallas import tpu as pltpu

_N = 20000
_ROWS = 160
_LANES = 128
_PAD = _ROWS * _LANES
_MAX_DET = 100
_IOU_THRESH = 0.5
_SCORE_THRESH = 0.05
_NEG = -1e9


def _nms_body(x1_ref, y1_ref, x2_ref, y2_ref, sc_ref, out_ref, s_ref):
    lane = jax.lax.broadcasted_iota(jnp.int32, (1, _LANES), 1)
    row_i = jax.lax.broadcasted_iota(jnp.int32, (_ROWS, _LANES), 0)
    col_i = jax.lax.broadcasted_iota(jnp.int32, (_ROWS, _LANES), 1)
    flat = row_i * _LANES + col_i

    scores = sc_ref[...]
    s = jnp.where(scores > _SCORE_THRESH, scores, jnp.float32(_NEG))
    s = jnp.where(flat < _N, s, -jnp.inf)
    s_ref[...] = s

    zvec = jnp.zeros((1, _LANES), jnp.float32)

    def cond(state):
        kc, fc, processed = state[0], state[1], state[2]
        return (kc < _MAX_DET) & (processed < _N)

    def body(state):
        (kc, fc, processed,
         kx1, ky1, kx2, ky2, ka, ks,
         fx1, fy1, fx2, fy2, fs) = state
        sarr = s_ref[...]
        m = jnp.max(sarr)
        j = jnp.min(jnp.where(sarr == m, flat, jnp.int32(_PAD)))
        r = j // _LANES
        c = j % _LANES

        def pick(ref):
            rowv = ref[pl.ds(r, 1), :]
            return jnp.max(jnp.where(lane == c, rowv, -jnp.inf))

        bx1 = pick(x1_ref)
        by1 = pick(y1_ref)
        bx2 = pick(x2_ref)
        by2 = pick(y2_ref)
        area = jnp.maximum(bx2 - bx1, 0.0) * jnp.maximum(by2 - by1, 0.0)

        xx1 = jnp.maximum(kx1, bx1)
        yy1 = jnp.maximum(ky1, by1)
        xx2 = jnp.minimum(kx2, bx2)
        yy2 = jnp.minimum(ky2, by2)
        inter = jnp.maximum(xx2 - xx1, 0.0) * jnp.maximum(yy2 - yy1, 0.0)
        iou = inter / (ka + area - inter + 1e-9)
        iou = jnp.where(lane < kc, iou, -1.0)
        suppressed = jnp.max(iou) > _IOU_THRESH
        valid = m > -1e8
        keep_it = valid & (~suppressed)

        sel_k = (lane == kc) & keep_it
        kx1 = jnp.where(sel_k, bx1, kx1)
        ky1 = jnp.where(sel_k, by1, ky1)
        kx2 = jnp.where(sel_k, bx2, kx2)
        ky2 = jnp.where(sel_k, by2, ky2)
        ka = jnp.where(sel_k, area, ka)
        ks = jnp.where(sel_k, m, ks)

        filler_slot = (~keep_it) & (fc < _MAX_DET)
        sel_f = (lane == fc) & filler_slot
        fx1 = jnp.where(sel_f, bx1, fx1)
        fy1 = jnp.where(sel_f, by1, fy1)
        fx2 = jnp.where(sel_f, bx2, fx2)
        fy2 = jnp.where(sel_f, by2, fy2)
        fs = jnp.where(sel_f, m, fs)

        kc = kc + keep_it.astype(jnp.int32)
        fc = fc + filler_slot.astype(jnp.int32)
        processed = processed + 1

        rowv = s_ref[pl.ds(r, 1), :]
        s_ref[pl.ds(r, 1), :] = jnp.where(lane == c, -jnp.inf, rowv)

        return (kc, fc, processed,
                kx1, ky1, kx2, ky2, ka, ks,
                fx1, fy1, fx2, fy2, fs)

    init = (jnp.int32(0), jnp.int32(0), jnp.int32(0),
            zvec, zvec, zvec, zvec, zvec, zvec,
            zvec, zvec, zvec, zvec, zvec)
    (kc, fc, processed,
     kx1, ky1, kx2, ky2, ka, ks,
     fx1, fy1, fx2, fy2, fs) = jax.lax.while_loop(cond, body, init)

    def no_fill(_):
        return (kx1, ky1, kx2, ky2, ks)

    def do_fill(_):
        def fill_body(i, carry):
            ox1, oy1, ox2, oy2, osc = carry
            from_kept = i < kc
            idx = jnp.where(from_kept, i, i - kc)
            sel_src = lane == idx

            def pickv(kv, fv):
                vk = jnp.max(jnp.where(sel_src, kv, -jnp.inf))
                vf = jnp.max(jnp.where(sel_src, fv, -jnp.inf))
                return jnp.where(from_kept, vk, vf)

            sel_dst = lane == i
            ox1 = jnp.where(sel_dst, pickv(kx1, fx1), ox1)
            oy1 = jnp.where(sel_dst, pickv(ky1, fy1), oy1)
            ox2 = jnp.where(sel_dst, pickv(kx2, fx2), ox2)
            oy2 = jnp.where(sel_dst, pickv(ky2, fy2), oy2)
            osc = jnp.where(sel_dst, pickv(ks, fs), osc)
            return (ox1, oy1, ox2, oy2, osc)

        return jax.lax.fori_loop(
            0, _MAX_DET, fill_body, (zvec, zvec, zvec, zvec, zvec))

    ox1, oy1, ox2, oy2, osc = jax.lax.cond(kc >= _MAX_DET, no_fill, do_fill,
                                           None)

    out_ref[...] = jnp.zeros((8, _LANES), jnp.float32)
    out_ref[0:1, :] = ox1
    out_ref[1:2, :] = oy1
    out_ref[2:3, :] = ox2
    out_ref[3:4, :] = oy2
    out_ref[4:5, :] = osc


def kernel(boxes, scores):
    def padded(col):
        return jnp.pad(col, (0, _PAD - _N)).reshape(_ROWS, _LANES)

    x1 = padded(boxes[:, 0])
    y1 = padded(boxes[:, 1])
    x2 = padded(boxes[:, 2])
    y2 = padded(boxes[:, 3])
    sc = padded(scores)

    out = pl.pallas_call(
        _nms_body,
        out_shape=jax.ShapeDtypeStruct((8, _LANES), jnp.float32),
        scratch_shapes=[pltpu.VMEM((_ROWS, _LANES), jnp.float32)],
    )(x1, y1, x2, y2, sc)

    return jnp.stack([out[0, :_MAX_DET], out[1, :_MAX_DET],
                      out[2, :_MAX_DET], out[3, :_MAX_DET],
                      out[4, :_MAX_DET]], axis=1)


# speedup vs baseline: 2188.4407x; 1.3394x over previous
import functools

import jax
import jax.numpy as jnp
from jax import lax
from jax.experimental import pallas as pl
from jax.experimental.pallas import tpu as pltpu
from jax.experimental.pallas import tpu_sc as plsc

_N = 20000
_L = 16
_C1 = 1264
_MAX_DET = 100
_KCAP = 112
_IOU_THRESH = 0.5
_SCORE_THRESH = 0.05
_NEG = -1e9


def _sc_body(x1_hbm, y1_hbm, x2_hbm, y2_hbm, sc_hbm, out_hbm,
             s_ref, x1_ref, y1_ref, x2_ref, y2_ref,
             l1_ref, l2_ref, l3_ref,
             kx1, ky1, kx2, ky2, ks, ka,
             fx1, fy1, fx2, fy2, fs):
    tile0 = (lax.axis_index("c") == 0) & (lax.axis_index("s") == 0)

    @pl.when(tile0)
    def _():
        iota = lax.broadcasted_iota(jnp.int32, (_L,), 0)
        lane0 = iota == 0
        ninf = jnp.full((_L,), -jnp.inf, jnp.float32)
        zero = jnp.zeros((_L,), jnp.float32)

        pltpu.sync_copy(x1_hbm, x1_ref)
        pltpu.sync_copy(y1_hbm, y1_ref)
        pltpu.sync_copy(x2_hbm, x2_ref)
        pltpu.sync_copy(y2_hbm, y2_ref)
        pltpu.sync_copy(sc_hbm, s_ref.at[pl.ds(0, _N)])

        for t in range((_C1 * _L - _N) // _L):
            s_ref[pl.ds(_N + t * _L, _L)] = ninf
        l1_ref[pl.ds(_C1, _L)] = ninf

        for t in range(_KCAP // _L):
            sl = pl.ds(t * _L, _L)
            kx1[sl] = zero
            ky1[sl] = zero
            kx2[sl] = zero
            ky2[sl] = zero
            ks[sl] = zero
            ka[sl] = zero
            fx1[sl] = zero
            fy1[sl] = zero
            fx2[sl] = zero
            fy2[sl] = zero
            fs[sl] = zero

        def thr_body(c, _):
            sl = pl.ds(c * _L, _L)
            v = s_ref[sl]
            s_ref[sl] = jnp.where(v > _SCORE_THRESH, v, jnp.float32(_NEG))
            return 0
        lax.fori_loop(0, _N // _L, thr_body, 0)

        def l1_body(k, _):
            base = k * (_L * _L)
            acc = plsc.load_gather(s_ref, [iota * _L + base])
            for t in range(1, _L):
                acc = jnp.maximum(
                    acc, plsc.load_gather(s_ref, [iota * _L + (base + t)]))
            l1_ref[pl.ds(k * _L, _L)] = acc
            return 0
        lax.fori_loop(0, _C1 // _L, l1_body, 0)

        def l2_body(k, _):
            base = k * (_L * _L)
            acc = plsc.load_gather(l1_ref, [iota * _L + base])
            for t in range(1, _L):
                acc = jnp.maximum(
                    acc, plsc.load_gather(l1_ref, [iota * _L + (base + t)]))
            l2_ref[pl.ds(k * _L, _L)] = acc
            return 0
        lax.fori_loop(0, 80 // _L, l2_body, 0)

        l3 = ninf
        for q in range(80 // _L):
            mq = jnp.max(l2_ref[pl.ds(q * _L, _L)])
            l3 = jnp.where(iota == q, jnp.full((_L,), mq), l3)
        l3_ref[...] = l3

        def cond(st):
            kc, fc, processed = st
            return (kc < _MAX_DET) & (processed < _N)

        def body(st):
            kc, fc, processed = st
            l3v = l3_ref[...]
            m = jnp.max(l3v)
            m_v = jnp.full((_L,), m)
            q_v = plsc.all_reduce_ffs(l3v == m_v)
            l2v = plsc.load_gather(l2_ref, [q_v * _L + iota])
            g_v = q_v * _L + plsc.all_reduce_ffs(l2v == m_v)
            l1v = plsc.load_gather(l1_ref, [g_v * _L + iota])
            c_v = g_v * _L + plsc.all_reduce_ffs(l1v == m_v)
            sv = plsc.load_gather(s_ref, [c_v * _L + iota])
            j_v = c_v * _L + plsc.all_reduce_ffs(sv == m_v)

            bx1 = plsc.load_gather(x1_ref, [j_v])
            by1 = plsc.load_gather(y1_ref, [j_v])
            bx2 = plsc.load_gather(x2_ref, [j_v])
            by2 = plsc.load_gather(y2_ref, [j_v])
            area = (jnp.maximum(bx2 - bx1, 0.0)
                    * jnp.maximum(by2 - by1, 0.0))

            acc = jnp.full((_L,), -1.0, jnp.float32)
            for kk in range(_KCAP // _L):
                sl = pl.ds(kk * _L, _L)
                xx1 = jnp.maximum(kx1[sl], bx1)
                yy1 = jnp.maximum(ky1[sl], by1)
                xx2 = jnp.minimum(kx2[sl], bx2)
                yy2 = jnp.minimum(ky2[sl], by2)
                inter = (jnp.maximum(xx2 - xx1, 0.0)
                         * jnp.maximum(yy2 - yy1, 0.0))
                iou = inter / (ka[sl] + area - inter + 1e-9)
                lanemask = (iota + kk * _L) < kc
                acc = jnp.maximum(acc, jnp.where(lanemask, iou, -1.0))
            suppressed = jnp.max(acc) > _IOU_THRESH
            valid = m > -1e8
            keep_it = valid & jnp.logical_not(suppressed)

            @pl.when(keep_it)
            def _():
                kc_v = jnp.full((_L,), kc)
                plsc.store_scatter(kx1, [kc_v], bx1, mask=lane0)
                plsc.store_scatter(ky1, [kc_v], by1, mask=lane0)
                plsc.store_scatter(kx2, [kc_v], bx2, mask=lane0)
                plsc.store_scatter(ky2, [kc_v], by2, mask=lane0)
                plsc.store_scatter(ks, [kc_v], m_v, mask=lane0)
                plsc.store_scatter(ka, [kc_v], area, mask=lane0)

            filler_slot = jnp.logical_not(keep_it) & (fc < _MAX_DET)

            @pl.when(filler_slot)
            def _():
                fc_v = jnp.full((_L,), fc)
                plsc.store_scatter(fx1, [fc_v], bx1, mask=lane0)
                plsc.store_scatter(fy1, [fc_v], by1, mask=lane0)
                plsc.store_scatter(fx2, [fc_v], bx2, mask=lane0)
                plsc.store_scatter(fy2, [fc_v], by2, mask=lane0)
                plsc.store_scatter(fs, [fc_v], m_v, mask=lane0)

            plsc.store_scatter(s_ref, [j_v], ninf, mask=lane0)
            sv2 = plsc.load_gather(s_ref, [c_v * _L + iota])
            plsc.store_scatter(l1_ref, [c_v],
                               jnp.full((_L,), jnp.max(sv2)), mask=lane0)
            l1v2 = plsc.load_gather(l1_ref, [g_v * _L + iota])
            plsc.store_scatter(l2_ref, [g_v],
                               jnp.full((_L,), jnp.max(l1v2)), mask=lane0)
            l2v2 = plsc.load_gather(l2_ref, [q_v * _L + iota])
            plsc.store_scatter(l3_ref, [q_v],
                               jnp.full((_L,), jnp.max(l2v2)), mask=lane0)

            kc = jnp.where(keep_it, kc + 1, kc)
            fc = jnp.where(filler_slot, fc + 1, fc)
            return (kc, fc, processed + 1)

        kc, fc, _ = lax.while_loop(
            cond, body, (jnp.int32(0), jnp.int32(0), jnp.int32(0)))

        def fcond(i):
            return i < _MAX_DET

        def fbody(i):
            src = jnp.full((_L,), i - kc)
            dst = jnp.full((_L,), i)
            for kref, fref in ((kx1, fx1), (ky1, fy1), (kx2, fx2),
                               (ky2, fy2), (ks, fs)):
                v = plsc.load_gather(fref, [src])
                plsc.store_scatter(kref, [dst], v, mask=lane0)
            return i + 1

        lax.while_loop(fcond, fbody, kc)

        pltpu.sync_copy(kx1, out_hbm.at[pl.ds(0 * _KCAP, _KCAP)])
        pltpu.sync_copy(ky1, out_hbm.at[pl.ds(1 * _KCAP, _KCAP)])
        pltpu.sync_copy(kx2, out_hbm.at[pl.ds(2 * _KCAP, _KCAP)])
        pltpu.sync_copy(ky2, out_hbm.at[pl.ds(3 * _KCAP, _KCAP)])
        pltpu.sync_copy(ks, out_hbm.at[pl.ds(4 * _KCAP, _KCAP)])


_sc_nms = functools.partial(
    pl.kernel,
    out_type=jax.ShapeDtypeStruct((5 * _KCAP,), jnp.float32),
    mesh=plsc.VectorSubcoreMesh(core_axis_name="c", subcore_axis_name="s"),
    compiler_params=pltpu.CompilerParams(needs_layout_passes=False),
    scratch_types=[
        pltpu.VMEM((_C1 * _L,), jnp.float32),
        pltpu.VMEM((_N,), jnp.float32),
        pltpu.VMEM((_N,), jnp.float32),
        pltpu.VMEM((_N,), jnp.float32),
        pltpu.VMEM((_N,), jnp.float32),
        pltpu.VMEM((_C1 + _L,), jnp.float32),
        pltpu.VMEM((80,), jnp.float32),
        pltpu.VMEM((_L,), jnp.float32),
        pltpu.VMEM((_KCAP,), jnp.float32),
        pltpu.VMEM((_KCAP,), jnp.float32),
        pltpu.VMEM((_KCAP,), jnp.float32),
        pltpu.VMEM((_KCAP,), jnp.float32),
        pltpu.VMEM((_KCAP,), jnp.float32),
        pltpu.VMEM((_KCAP,), jnp.float32),
        pltpu.VMEM((_KCAP,), jnp.float32),
        pltpu.VMEM((_KCAP,), jnp.float32),
        pltpu.VMEM((_KCAP,), jnp.float32),
        pltpu.VMEM((_KCAP,), jnp.float32),
        pltpu.VMEM((_KCAP,), jnp.float32),
    ],
)(_sc_body)


def kernel(boxes, scores):
    x1 = boxes[:, 0]
    y1 = boxes[:, 1]
    x2 = boxes[:, 2]
    y2 = boxes[:, 3]
    out = _sc_nms(x1, y1, x2, y2, scores).reshape(5, _KCAP)
    return jnp.stack([out[0, :_MAX_DET], out[1, :_MAX_DET],
                      out[2, :_MAX_DET], out[3, :_MAX_DET],
                      out[4, :_MAX_DET]], axis=1)
